# baseline (speedup 1.0000x reference)
import functools

import jax
import jax.numpy as jnp
from jax import lax
from jax.experimental import pallas as pl
from jax.experimental.pallas import tpu as pltpu
from jax.experimental.pallas import tpu_sc as plsc

N = 10000
D_IN = 128
H1 = 200
H2 = 100
N_CLASSES = 40

NC, NS, L = 2, 16, 16
NW = NC * NS
B = 128
NPAD = 10112
ROWS_PER_TILE = NPAD // NS
F = 128
FH = F // NC
DEGW = 16
NBUF = 6
PF = 4

_mesh = plsc.VectorSubcoreMesh(core_axis_name="c", subcore_axis_name="s")


def _copy_idx_row(src2d, j, dst1d):
    for k in range(B // L):
        dst1d[pl.ds(k * L, L)] = src2d[j, pl.ds(k * L, L)]


def _fill_rows(ref, n_rows, width, val):
    def body(i, _):
        r = i // (width // L)
        c = (i % (width // L)) * L
        ref[r, pl.ds(c, L)] = jnp.full((L,), val, jnp.float32)
        return 0
    lax.fori_loop(0, n_rows * (width // L), body, 0)


def _make_deg_kernel(bpd):
    @functools.partial(
        pl.kernel,
        out_type=jax.ShapeDtypeStruct((NC, NPAD, DEGW), jnp.float32),
        mesh=_mesh,
        scratch_types=[
            pltpu.VMEM((bpd, B), jnp.int32),
            pltpu.VMEM((B,), jnp.int32),
            pltpu.VMEM((B, DEGW), jnp.float32),
            pltpu.VMEM((ROWS_PER_TILE, DEGW), jnp.float32),
            pltpu.VMEM_SHARED((NPAD, DEGW), jnp.float32),
        ],
        compiler_params=pltpu.CompilerParams(use_tc_tiling_on_sc=False),
    )
    def deg_kernel(dst_hbm, deg_hbm, dst_v, dcur, ones_v, bounce_v, hist_sh):
        cid = lax.axis_index("c")
        sid = lax.axis_index("s")
        wid = sid * NC + cid
        _fill_rows(ones_v, B, DEGW, 1.0)
        _fill_rows(bounce_v, ROWS_PER_TILE, DEGW, 0.0)
        base = sid * ROWS_PER_TILE
        pltpu.sync_copy(bounce_v, hist_sh.at[pl.ds(base, ROWS_PER_TILE)])
        pltpu.sync_copy(dst_hbm.at[wid], dst_v)
        plsc.subcore_barrier()

        def body(j, _):
            _copy_idx_row(dst_v, j, dcur)
            pltpu.sync_copy(ones_v, hist_sh.at[dcur], add=True)
            return 0
        lax.fori_loop(0, bpd, body, 0)

        plsc.subcore_barrier()
        pltpu.sync_copy(hist_sh.at[pl.ds(base, ROWS_PER_TILE)], bounce_v)
        pltpu.sync_copy(bounce_v, deg_hbm.at[cid, pl.ds(base, ROWS_PER_TILE)])

    return deg_kernel


def _make_agg_kernel(bp):
    @functools.partial(
        pl.kernel,
        out_type=jax.ShapeDtypeStruct((NC, NPAD, FH), jnp.float32),
        mesh=_mesh,
        scratch_types=[
            pltpu.VMEM((bp, B), jnp.int32),
            pltpu.VMEM((bp, B), jnp.int32),
            pltpu.VMEM((NBUF, B), jnp.int32),
            pltpu.VMEM((NBUF, B, FH), jnp.float32),
            pltpu.VMEM_SHARED((NPAD, FH), jnp.float32),
            pltpu.SemaphoreType.DMA,
            pltpu.SemaphoreType.DMA,
        ],
        compiler_params=pltpu.CompilerParams(use_tc_tiling_on_sc=False),
    )
    def agg_kernel(y_hbm, src_hbm, dst_hbm, out_hbm,
                   src_v, dst_v, dring, rows_v, acc_sh, gsem, ssem):
        cid = lax.axis_index("c")
        sid = lax.axis_index("s")

        _fill_rows(rows_v.at[0], B, FH, 0.0)
        base = sid * ROWS_PER_TILE
        n_full = ROWS_PER_TILE // B
        for cchunk in range(n_full):
            pltpu.sync_copy(rows_v.at[0],
                            acc_sh.at[pl.ds(base + cchunk * B, B)])
        rem = ROWS_PER_TILE - n_full * B
        if rem:
            pltpu.sync_copy(rows_v.at[0, pl.ds(0, rem)],
                            acc_sh.at[pl.ds(base + n_full * B, rem)])

        pltpu.sync_copy(src_hbm.at[cid, sid], src_v)
        pltpu.sync_copy(dst_hbm.at[sid], dst_v)
        plsc.subcore_barrier()

        def _fire_gather(j):
            pltpu.async_copy(y_hbm.at[src_v.at[j]], rows_v.at[j % NBUF], gsem)

        def _wait_gather(j):
            pltpu.make_async_copy(y_hbm.at[src_v.at[j]],
                                  rows_v.at[j % NBUF], gsem).wait()

        def _fire_scatter(j):
            pltpu.async_copy(rows_v.at[j % NBUF], acc_sh.at[dring.at[j % NBUF]],
                             ssem, add=True)

        def _wait_scatter(j):
            pltpu.make_async_copy(rows_v.at[j % NBUF],
                                  acc_sh.at[dring.at[j % NBUF]], ssem).wait()

        for t in range(PF):
            _fire_gather(t)

        def body(j, _):
            @pl.when(j + PF < bp)
            def _prefetch():
                @pl.when(j >= NBUF - PF)
                def _drain():
                    _wait_scatter(j - (NBUF - PF))
                _fire_gather(j + PF)
            _wait_gather(j)
            _copy_idx_row(dst_v, j, dring.at[j % NBUF])
            _fire_scatter(j)
            return 0
        lax.fori_loop(0, bp, body, 0)
        for t in range(NBUF):
            _wait_scatter(bp - NBUF + t)

        plsc.subcore_barrier()
        for cchunk in range(n_full + (1 if rem else 0)):
            lo = base + cchunk * B
            nr = B if cchunk < n_full else rem
            pltpu.sync_copy(acc_sh.at[pl.ds(lo, nr)],
                            rows_v.at[1, pl.ds(0, nr)])
            pltpu.sync_copy(rows_v.at[1, pl.ds(0, nr)],
                            out_hbm.at[cid, pl.ds(lo, nr)])

    return agg_kernel


_BM = 2528


def _dinv_block(degp_ref):
    d = degp_ref[0, :, 0:1] + degp_ref[1, :, 0:1] + 1.0
    return lax.rsqrt(d)


def _scale_kernel(x_ref, degp_ref, y_ref):
    y = x_ref[...] * _dinv_block(degp_ref)
    y_ref[0] = y[:, :FH]
    y_ref[1] = y[:, FH:]


def _layer1_kernel(agg_ref, y_ref, degp_ref, w1_ref, b1_ref, w2_ref, o_ref):
    dinv = _dinv_block(degp_ref)
    sy = jnp.concatenate([agg_ref[0] + y_ref[0], agg_ref[1] + y_ref[1]],
                         axis=1)
    z = sy * dinv
    h1 = jax.nn.relu(
        jnp.dot(z, w1_ref[...], preferred_element_type=jnp.float32)
        + b1_ref[...])
    y2 = jnp.dot(h1, w2_ref[...], preferred_element_type=jnp.float32) * dinv
    o_ref[0] = y2[:, :FH]
    o_ref[1] = y2[:, FH:]


def _layer3_kernel(agg_ref, y_ref, degp_ref, b2_ref, w3_ref, b3_ref, o_ref):
    dinv = _dinv_block(degp_ref)
    sy = jnp.concatenate([agg_ref[0] + y_ref[0], agg_ref[1] + y_ref[1]],
                         axis=1)
    h2 = jax.nn.relu(sy * dinv + b2_ref[...])
    o_ref[...] = (jnp.dot(h2, w3_ref[...], preferred_element_type=jnp.float32)
                  + b3_ref[...])


_degp_spec = pl.BlockSpec((NC, _BM, DEGW), lambda i: (0, i, 0))
_half_spec = pl.BlockSpec((NC, _BM, FH), lambda i: (0, i, 0))


def _node_spec(width):
    return pl.BlockSpec((_BM, width), lambda i: (i, 0))


def _full(a, b):
    return pl.BlockSpec((a, b), lambda i: (0, 0))


def kernel(x, edge_index, W1, b1, W2, b2, W3, b3):
    e = edge_index.shape[1]
    e_pad = ((e + NW * B - 1) // (NW * B)) * (NW * B)
    bp = e_pad // (NS * B)
    bpd = e_pad // (NW * B)

    src = edge_index[0].astype(jnp.int32)
    dst = edge_index[1].astype(jnp.int32)
    pad_idx = N + (jnp.arange(e_pad - e, dtype=jnp.int32) % (NPAD - N))
    src_p = jnp.concatenate([src, pad_idx]).reshape(NS, bp, B)
    src_p2 = jnp.stack([src_p, src_p + NPAD])
    dst_p = jnp.concatenate([dst, pad_idx]).reshape(NS, bp, B)
    dst_pd = dst_p.reshape(NW, bpd, B)

    xpad = jnp.pad(x, ((0, NPAD - N), (0, 0)))
    w1p = jnp.pad(W1, ((0, 0), (0, 256 - H1)))
    b1p = jnp.pad(b1, (0, 256 - H1)).reshape(1, 256)
    w2p = jnp.pad(W2, ((0, 256 - H1), (0, F - H2)))
    b2p = jnp.pad(b2, (0, F - H2)).reshape(1, F)
    w3p = jnp.pad(W3, ((0, F - H2), (0, F - N_CLASSES)))
    b3p = jnp.pad(b3, (0, F - N_CLASSES)).reshape(1, F)

    degp = _make_deg_kernel(bpd)(dst_pd)

    grid = NPAD // _BM
    agg = _make_agg_kernel(bp)

    y1 = pl.pallas_call(
        _scale_kernel,
        out_shape=jax.ShapeDtypeStruct((NC, NPAD, FH), jnp.float32),
        grid=(grid,),
        in_specs=[_node_spec(F), _degp_spec],
        out_specs=_half_spec,
    )(xpad, degp)

    agg1 = agg(y1.reshape(NC * NPAD, FH), src_p2, dst_p)

    y2 = pl.pallas_call(
        _layer1_kernel,
        out_shape=jax.ShapeDtypeStruct((NC, NPAD, FH), jnp.float32),
        grid=(grid,),
        in_specs=[_half_spec, _half_spec, _degp_spec,
                  _full(F, 256), _full(1, 256), _full(256, F)],
        out_specs=_half_spec,
    )(agg1, y1, degp, w1p, b1p, w2p)

    agg2 = agg(y2.reshape(NC * NPAD, FH), src_p2, dst_p)

    outp = pl.pallas_call(
        _layer3_kernel,
        out_shape=jax.ShapeDtypeStruct((NPAD, F), jnp.float32),
        grid=(grid,),
        in_specs=[_half_spec, _half_spec, _degp_spec,
                  _full(1, F), _full(F, F), _full(1, F)],
        out_specs=_node_spec(F),
    )(agg2, y2, degp, b2p, w3p, b3p)

    return outp[:N, :N_CLASSES]


# speedup vs baseline: 1.0186x; 1.0186x over previous
import functools

import jax
import jax.numpy as jnp
from jax import lax
from jax.experimental import pallas as pl
from jax.experimental.pallas import tpu as pltpu
from jax.experimental.pallas import tpu_sc as plsc

N = 10000
D_IN = 128
H1 = 200
H2 = 100
N_CLASSES = 40

NC, NS, L = 2, 16, 16
NW = NC * NS
B = 128
NPAD = 10112
ROWS_PER_TILE = NPAD // NS
F = 128
FH = F // NC
DEGW = 16
NBUF = 6
PF = 4

_mesh = plsc.VectorSubcoreMesh(core_axis_name="c", subcore_axis_name="s")


def _copy_idx_row(src2d, j, dst1d):
    for k in range(B // L):
        dst1d[pl.ds(k * L, L)] = src2d[j, pl.ds(k * L, L)]


def _fill_rows(ref, n_rows, width, val):
    def body(i, _):
        r = i // (width // L)
        c = (i % (width // L)) * L
        ref[r, pl.ds(c, L)] = jnp.full((L,), val, jnp.float32)
        return 0
    lax.fori_loop(0, n_rows * (width // L), body, 0)


def _make_deg_kernel(bp):
    bpd = bp // NC

    @functools.partial(
        pl.kernel,
        out_type=jax.ShapeDtypeStruct((NC, NPAD, DEGW), jnp.float32),
        mesh=_mesh,
        scratch_types=[
            pltpu.VMEM((bpd, B), jnp.int32),
            pltpu.VMEM((4, B), jnp.int32),
            pltpu.VMEM((B, DEGW), jnp.float32),
            pltpu.VMEM((ROWS_PER_TILE, DEGW), jnp.float32),
            pltpu.VMEM_SHARED((NPAD, DEGW), jnp.float32),
            pltpu.SemaphoreType.DMA,
        ],
        compiler_params=pltpu.CompilerParams(use_tc_tiling_on_sc=False),
    )
    def deg_kernel(dst_hbm, deg_hbm, dst_v, dring, ones_v, bounce_v, hist_sh,
                   ssem):
        cid = lax.axis_index("c")
        sid = lax.axis_index("s")
        _fill_rows(ones_v, B, DEGW, 1.0)
        _fill_rows(bounce_v, ROWS_PER_TILE, DEGW, 0.0)
        base = sid * ROWS_PER_TILE
        pltpu.sync_copy(bounce_v, hist_sh.at[pl.ds(base, ROWS_PER_TILE)])
        pltpu.sync_copy(dst_hbm.at[sid, pl.ds(cid * bpd, bpd)], dst_v)
        plsc.subcore_barrier()

        def _wait_scatter(j):
            pltpu.make_async_copy(ones_v, hist_sh.at[dring.at[j % 4]],
                                  ssem).wait()

        def body(j, _):
            @pl.when(j >= 4)
            def _drain():
                _wait_scatter(j - 4)
            _copy_idx_row(dst_v, j, dring.at[j % 4])
            pltpu.async_copy(ones_v, hist_sh.at[dring.at[j % 4]], ssem,
                             add=True)
            return 0
        lax.fori_loop(0, bpd, body, 0)
        for t in range(4):
            _wait_scatter(bpd - 4 + t)

        plsc.subcore_barrier()
        pltpu.sync_copy(hist_sh.at[pl.ds(base, ROWS_PER_TILE)], bounce_v)
        pltpu.sync_copy(bounce_v, deg_hbm.at[cid, pl.ds(base, ROWS_PER_TILE)])

    return deg_kernel


def _make_agg_kernel(bp):
    @functools.partial(
        pl.kernel,
        out_type=jax.ShapeDtypeStruct((NC, NPAD, FH), jnp.float32),
        mesh=_mesh,
        scratch_types=[
            pltpu.VMEM((bp, B), jnp.int32),
            pltpu.VMEM((bp, B), jnp.int32),
            pltpu.VMEM((NBUF, B), jnp.int32),
            pltpu.VMEM((NBUF, B, FH), jnp.float32),
            pltpu.VMEM_SHARED((NPAD, FH), jnp.float32),
            pltpu.SemaphoreType.DMA,
            pltpu.SemaphoreType.DMA,
        ],
        compiler_params=pltpu.CompilerParams(use_tc_tiling_on_sc=False),
    )
    def agg_kernel(y_hbm, src_hbm, dst_hbm, out_hbm,
                   src_v, dst_v, dring, rows_v, acc_sh, gsem, ssem):
        cid = lax.axis_index("c")
        sid = lax.axis_index("s")

        _fill_rows(rows_v.at[0], B, FH, 0.0)
        base = sid * ROWS_PER_TILE
        n_full = ROWS_PER_TILE // B
        for cchunk in range(n_full):
            pltpu.sync_copy(rows_v.at[0],
                            acc_sh.at[pl.ds(base + cchunk * B, B)])
        rem = ROWS_PER_TILE - n_full * B
        if rem:
            pltpu.sync_copy(rows_v.at[0, pl.ds(0, rem)],
                            acc_sh.at[pl.ds(base + n_full * B, rem)])

        pltpu.sync_copy(src_hbm.at[cid, sid], src_v)
        pltpu.sync_copy(dst_hbm.at[sid], dst_v)
        plsc.subcore_barrier()

        def _fire_gather(j):
            pltpu.async_copy(y_hbm.at[src_v.at[j]], rows_v.at[j % NBUF], gsem)

        def _wait_gather(j):
            pltpu.make_async_copy(y_hbm.at[src_v.at[j]],
                                  rows_v.at[j % NBUF], gsem).wait()

        def _fire_scatter(j):
            pltpu.async_copy(rows_v.at[j % NBUF], acc_sh.at[dring.at[j % NBUF]],
                             ssem, add=True)

        def _wait_scatter(j):
            pltpu.make_async_copy(rows_v.at[j % NBUF],
                                  acc_sh.at[dring.at[j % NBUF]], ssem).wait()

        for t in range(PF):
            _fire_gather(t)

        def body(j, _):
            @pl.when(j + PF < bp)
            def _prefetch():
                @pl.when(j >= NBUF - PF)
                def _drain():
                    _wait_scatter(j - (NBUF - PF))
                _fire_gather(j + PF)
            _wait_gather(j)
            _copy_idx_row(dst_v, j, dring.at[j % NBUF])
            _fire_scatter(j)
            return 0
        lax.fori_loop(0, bp, body, 0)
        for t in range(NBUF):
            _wait_scatter(bp - NBUF + t)

        plsc.subcore_barrier()
        for cchunk in range(n_full + (1 if rem else 0)):
            lo = base + cchunk * B
            nr = B if cchunk < n_full else rem
            pltpu.sync_copy(acc_sh.at[pl.ds(lo, nr)],
                            rows_v.at[1, pl.ds(0, nr)])
            pltpu.sync_copy(rows_v.at[1, pl.ds(0, nr)],
                            out_hbm.at[cid, pl.ds(lo, nr)])

    return agg_kernel


_BM = 2528


def _dinv_block(degp_ref):
    d = degp_ref[0, :, 0:1] + degp_ref[1, :, 0:1] + 1.0
    return lax.rsqrt(d)


def _scale_kernel(x_ref, degp_ref, y_ref):
    y = x_ref[...] * _dinv_block(degp_ref)
    y_ref[0] = y[:, :FH]
    y_ref[1] = y[:, FH:]


def _layer1_kernel(agg_ref, y_ref, degp_ref, w1_ref, b1_ref, w2_ref, o_ref):
    dinv = _dinv_block(degp_ref)
    sy = jnp.concatenate([agg_ref[0] + y_ref[0], agg_ref[1] + y_ref[1]],
                         axis=1)
    z = (sy * dinv).astype(jnp.bfloat16)
    h1 = jax.nn.relu(
        jnp.dot(z, w1_ref[...], preferred_element_type=jnp.float32)
        + b1_ref[...]).astype(jnp.bfloat16)
    y2 = jnp.dot(h1, w2_ref[...], preferred_element_type=jnp.float32) * dinv
    o_ref[0] = y2[:, :FH]
    o_ref[1] = y2[:, FH:]


def _layer3_kernel(agg_ref, y_ref, degp_ref, b2_ref, w3_ref, b3_ref, o_ref):
    dinv = _dinv_block(degp_ref)
    sy = jnp.concatenate([agg_ref[0] + y_ref[0], agg_ref[1] + y_ref[1]],
                         axis=1)
    h2 = jax.nn.relu(sy * dinv + b2_ref[...]).astype(jnp.bfloat16)
    o_ref[...] = (jnp.dot(h2, w3_ref[...], preferred_element_type=jnp.float32)
                  + b3_ref[...])


_degp_spec = pl.BlockSpec((NC, _BM, DEGW), lambda i: (0, i, 0))
_half_spec = pl.BlockSpec((NC, _BM, FH), lambda i: (0, i, 0))


def _node_spec(width):
    return pl.BlockSpec((_BM, width), lambda i: (i, 0))


def _full(a, b):
    return pl.BlockSpec((a, b), lambda i: (0, 0))


def kernel(x, edge_index, W1, b1, W2, b2, W3, b3):
    e = edge_index.shape[1]
    e_pad = ((e + NW * B - 1) // (NW * B)) * (NW * B)
    bp = e_pad // (NS * B)

    src = edge_index[0].astype(jnp.int32)
    dst = edge_index[1].astype(jnp.int32)
    pad_idx = N + (jnp.arange(e_pad - e, dtype=jnp.int32) % (NPAD - N))
    src_p = jnp.concatenate([src, pad_idx]).reshape(NS, bp, B)
    src_p2 = jnp.stack([src_p, src_p + NPAD])
    dst_p = jnp.concatenate([dst, pad_idx]).reshape(NS, bp, B)

    xpad = jnp.pad(x, ((0, NPAD - N), (0, 0)))
    w1p = jnp.pad(W1, ((0, 0), (0, 256 - H1))).astype(jnp.bfloat16)
    b1p = jnp.pad(b1, (0, 256 - H1)).reshape(1, 256)
    w2p = jnp.pad(W2, ((0, 256 - H1), (0, F - H2))).astype(jnp.bfloat16)
    b2p = jnp.pad(b2, (0, F - H2)).reshape(1, F)
    w3p = jnp.pad(W3, ((0, F - H2), (0, F - N_CLASSES))).astype(jnp.bfloat16)
    b3p = jnp.pad(b3, (0, F - N_CLASSES)).reshape(1, F)

    degp = _make_deg_kernel(bp)(dst_p)

    grid = NPAD // _BM
    agg = _make_agg_kernel(bp)

    y1 = pl.pallas_call(
        _scale_kernel,
        out_shape=jax.ShapeDtypeStruct((NC, NPAD, FH), jnp.float32),
        grid=(grid,),
        in_specs=[_node_spec(F), _degp_spec],
        out_specs=_half_spec,
    )(xpad, degp)

    agg1 = agg(y1.reshape(NC * NPAD, FH), src_p2, dst_p)

    y2 = pl.pallas_call(
        _layer1_kernel,
        out_shape=jax.ShapeDtypeStruct((NC, NPAD, FH), jnp.float32),
        grid=(grid,),
        in_specs=[_half_spec, _half_spec, _degp_spec,
                  _full(F, 256), _full(1, 256), _full(256, F)],
        out_specs=_half_spec,
    )(agg1, y1, degp, w1p, b1p, w2p)

    agg2 = agg(y2.reshape(NC * NPAD, FH), src_p2, dst_p)

    outp = pl.pallas_call(
        _layer3_kernel,
        out_shape=jax.ShapeDtypeStruct((NPAD, F), jnp.float32),
        grid=(grid,),
        in_specs=[_half_spec, _half_spec, _degp_spec,
                  _full(1, F), _full(F, F), _full(1, F)],
        out_specs=_node_spec(F),
    )(agg2, y2, degp, b2p, w3p, b3p)

    return outp[:N, :N_CLASSES]


# speedup vs baseline: 1.0198x; 1.0012x over previous
import functools

import jax
import jax.numpy as jnp
from jax import lax
from jax.experimental import pallas as pl
from jax.experimental.pallas import tpu as pltpu
from jax.experimental.pallas import tpu_sc as plsc

N = 10000
D_IN = 128
H1 = 200
H2 = 100
N_CLASSES = 40

NC, NS, L = 2, 16, 16
NW = NC * NS
B = 128
NPAD = 10112
ROWS_PER_TILE = NPAD // NS
F = 128
FH = F // NC
DEGW = 16
NBUF = 6
PF = 4

def _mesh():
    return plsc.VectorSubcoreMesh(core_axis_name="c", subcore_axis_name="s",
                                  num_cores=NC, num_subcores=NS)


def _copy_idx_row(src2d, j, dst1d):
    for k in range(B // L):
        dst1d[pl.ds(k * L, L)] = src2d[j, pl.ds(k * L, L)]


def _fill_rows(ref, n_rows, width, val):
    def body(i, _):
        r = i // (width // L)
        c = (i % (width // L)) * L
        ref[r, pl.ds(c, L)] = jnp.full((L,), val, jnp.float32)
        return 0
    lax.fori_loop(0, n_rows * (width // L), body, 0)


def _make_deg_kernel(bp):
    bpd = bp // NC

    @functools.partial(
        pl.kernel,
        out_type=jax.ShapeDtypeStruct((NC, NPAD, DEGW), jnp.float32),
        mesh=_mesh(),
        scratch_types=[
            pltpu.VMEM((bpd, B), jnp.int32),
            pltpu.VMEM((4, B), jnp.int32),
            pltpu.VMEM((B, DEGW), jnp.float32),
            pltpu.VMEM((ROWS_PER_TILE, DEGW), jnp.float32),
            pltpu.VMEM_SHARED((NPAD, DEGW), jnp.float32),
            pltpu.SemaphoreType.DMA,
        ],
        compiler_params=pltpu.CompilerParams(use_tc_tiling_on_sc=False),
    )
    def deg_kernel(dst_hbm, deg_hbm, dst_v, dring, ones_v, bounce_v, hist_sh,
                   ssem):
        cid = lax.axis_index("c")
        sid = lax.axis_index("s")
        _fill_rows(ones_v, B, DEGW, 1.0)
        _fill_rows(bounce_v, ROWS_PER_TILE, DEGW, 0.0)
        base = sid * ROWS_PER_TILE
        pltpu.sync_copy(bounce_v, hist_sh.at[pl.ds(base, ROWS_PER_TILE)])
        pltpu.sync_copy(dst_hbm.at[sid, pl.ds(cid * bpd, bpd)], dst_v)
        plsc.subcore_barrier()

        def _wait_scatter(j):
            pltpu.make_async_copy(ones_v, hist_sh.at[dring.at[j % 4]],
                                  ssem).wait()

        def body(j, _):
            @pl.when(j >= 4)
            def _drain():
                _wait_scatter(j - 4)
            _copy_idx_row(dst_v, j, dring.at[j % 4])
            pltpu.async_copy(ones_v, hist_sh.at[dring.at[j % 4]], ssem,
                             add=True)
            return 0
        lax.fori_loop(0, bpd, body, 0)
        for t in range(4):
            _wait_scatter(bpd - 4 + t)

        plsc.subcore_barrier()
        pltpu.sync_copy(hist_sh.at[pl.ds(base, ROWS_PER_TILE)], bounce_v)
        pltpu.sync_copy(bounce_v, deg_hbm.at[cid, pl.ds(base, ROWS_PER_TILE)])

    return deg_kernel


def _make_agg_kernel(bp):
    @functools.partial(
        pl.kernel,
        out_type=jax.ShapeDtypeStruct((NC, NPAD, FH), jnp.float32),
        mesh=_mesh(),
        scratch_types=[
            pltpu.VMEM((bp, B), jnp.int32),
            pltpu.VMEM((bp, B), jnp.int32),
            pltpu.VMEM((NBUF, B), jnp.int32),
            pltpu.VMEM((NBUF, B, FH), jnp.float32),
            pltpu.VMEM_SHARED((NPAD, FH), jnp.float32),
            pltpu.SemaphoreType.DMA,
            pltpu.SemaphoreType.DMA,
        ],
        compiler_params=pltpu.CompilerParams(use_tc_tiling_on_sc=False),
    )
    def agg_kernel(y_hbm, src_hbm, dst_hbm, out_hbm,
                   src_v, dst_v, dring, rows_v, acc_sh, gsem, ssem):
        cid = lax.axis_index("c")
        sid = lax.axis_index("s")

        _fill_rows(rows_v.at[0], B, FH, 0.0)
        base = sid * ROWS_PER_TILE
        n_full = ROWS_PER_TILE // B
        for cchunk in range(n_full):
            pltpu.sync_copy(rows_v.at[0],
                            acc_sh.at[pl.ds(base + cchunk * B, B)])
        rem = ROWS_PER_TILE - n_full * B
        if rem:
            pltpu.sync_copy(rows_v.at[0, pl.ds(0, rem)],
                            acc_sh.at[pl.ds(base + n_full * B, rem)])

        pltpu.sync_copy(src_hbm.at[cid, sid], src_v)
        pltpu.sync_copy(dst_hbm.at[sid], dst_v)
        plsc.subcore_barrier()

        def _fire_gather(j):
            pltpu.async_copy(y_hbm.at[src_v.at[j]], rows_v.at[j % NBUF], gsem)

        def _wait_gather(j):
            pltpu.make_async_copy(y_hbm.at[src_v.at[j]],
                                  rows_v.at[j % NBUF], gsem).wait()

        def _fire_scatter(j):
            pltpu.async_copy(rows_v.at[j % NBUF], acc_sh.at[dring.at[j % NBUF]],
                             ssem, add=True)

        def _wait_scatter(j):
            pltpu.make_async_copy(rows_v.at[j % NBUF],
                                  acc_sh.at[dring.at[j % NBUF]], ssem).wait()

        for t in range(PF):
            _fire_gather(t)

        def body(j, _):
            @pl.when(j + PF < bp)
            def _prefetch():
                @pl.when(j >= NBUF - PF)
                def _drain():
                    _wait_scatter(j - (NBUF - PF))
                _fire_gather(j + PF)
            _wait_gather(j)
            _copy_idx_row(dst_v, j, dring.at[j % NBUF])
            _fire_scatter(j)
            return 0
        lax.fori_loop(0, bp, body, 0)
        for t in range(NBUF):
            _wait_scatter(bp - NBUF + t)

        plsc.subcore_barrier()
        for cchunk in range(n_full + (1 if rem else 0)):
            lo = base + cchunk * B
            nr = B if cchunk < n_full else rem
            pltpu.sync_copy(acc_sh.at[pl.ds(lo, nr)],
                            rows_v.at[1, pl.ds(0, nr)])
            pltpu.sync_copy(rows_v.at[1, pl.ds(0, nr)],
                            out_hbm.at[cid, pl.ds(lo, nr)])

    return agg_kernel


_BM = 2528


def _dinv_block(degp_ref):
    d = degp_ref[0, :, 0:1] + degp_ref[1, :, 0:1] + 1.0
    return lax.rsqrt(d)


def _scale_kernel(x_ref, degp_ref, y_ref):
    y = x_ref[...] * _dinv_block(degp_ref)
    y_ref[0] = y[:, :FH]
    y_ref[1] = y[:, FH:]


def _layer1_kernel(agg_ref, y_ref, degp_ref, w1_ref, b1_ref, w2_ref, o_ref):
    dinv = _dinv_block(degp_ref)
    sy = jnp.concatenate([agg_ref[0] + y_ref[0], agg_ref[1] + y_ref[1]],
                         axis=1)
    z = (sy * dinv).astype(jnp.bfloat16)
    h1 = jax.nn.relu(
        jnp.dot(z, w1_ref[...], preferred_element_type=jnp.float32)
        + b1_ref[...]).astype(jnp.bfloat16)
    y2 = jnp.dot(h1, w2_ref[...], preferred_element_type=jnp.float32) * dinv
    o_ref[0] = y2[:, :FH]
    o_ref[1] = y2[:, FH:]


def _layer3_kernel(agg_ref, y_ref, degp_ref, b2_ref, w3_ref, b3_ref, o_ref):
    dinv = _dinv_block(degp_ref)
    sy = jnp.concatenate([agg_ref[0] + y_ref[0], agg_ref[1] + y_ref[1]],
                         axis=1)
    h2 = jax.nn.relu(sy * dinv + b2_ref[...]).astype(jnp.bfloat16)
    o_ref[...] = (jnp.dot(h2, w3_ref[...], preferred_element_type=jnp.float32)
                  + b3_ref[...])


_degp_spec = pl.BlockSpec((NC, _BM, DEGW), lambda i: (0, i, 0))
_half_spec = pl.BlockSpec((NC, _BM, FH), lambda i: (0, i, 0))


def _node_spec(width):
    return pl.BlockSpec((_BM, width), lambda i: (i, 0))


def _full(a, b):
    return pl.BlockSpec((a, b), lambda i: (0, 0))


def kernel(x, edge_index, W1, b1, W2, b2, W3, b3):
    e = edge_index.shape[1]
    e_pad = ((e + NW * B - 1) // (NW * B)) * (NW * B)
    bp = e_pad // (NS * B)

    src = edge_index[0].astype(jnp.int32)
    dst = edge_index[1].astype(jnp.int32)
    pad_idx = N + (jnp.arange(e_pad - e, dtype=jnp.int32) % (NPAD - N))
    src_p = jnp.concatenate([src, pad_idx]).reshape(NS, bp, B)
    src_p2 = jnp.stack([src_p, src_p + NPAD])
    dst_p = jnp.concatenate([dst, pad_idx]).reshape(NS, bp, B)

    xpad = jnp.pad(x, ((0, NPAD - N), (0, 0)))
    w1p = jnp.pad(W1, ((0, 0), (0, 256 - H1))).astype(jnp.bfloat16)
    b1p = jnp.pad(b1, (0, 256 - H1)).reshape(1, 256)
    w2p = jnp.pad(W2, ((0, 256 - H1), (0, F - H2))).astype(jnp.bfloat16)
    b2p = jnp.pad(b2, (0, F - H2)).reshape(1, F)
    w3p = jnp.pad(W3, ((0, F - H2), (0, F - N_CLASSES))).astype(jnp.bfloat16)
    b3p = jnp.pad(b3, (0, F - N_CLASSES)).reshape(1, F)

    degp = _make_deg_kernel(bp)(dst_p)

    grid = NPAD // _BM
    agg = _make_agg_kernel(bp)

    y1 = pl.pallas_call(
        _scale_kernel,
        out_shape=jax.ShapeDtypeStruct((NC, NPAD, FH), jnp.float32),
        grid=(grid,),
        in_specs=[_node_spec(F), _degp_spec],
        out_specs=_half_spec,
    )(xpad, degp)

    agg1 = agg(y1.reshape(NC * NPAD, FH), src_p2, dst_p)

    y2 = pl.pallas_call(
        _layer1_kernel,
        out_shape=jax.ShapeDtypeStruct((NC, NPAD, FH), jnp.float32),
        grid=(grid,),
        in_specs=[_half_spec, _half_spec, _degp_spec,
                  _full(F, 256), _full(1, 256), _full(256, F)],
        out_specs=_half_spec,
    )(agg1, y1, degp, w1p, b1p, w2p)

    agg2 = agg(y2.reshape(NC * NPAD, FH), src_p2, dst_p)

    outp = pl.pallas_call(
        _layer3_kernel,
        out_shape=jax.ShapeDtypeStruct((NPAD, F), jnp.float32),
        grid=(grid,),
        in_specs=[_half_spec, _half_spec, _degp_spec,
                  _full(1, F), _full(F, F), _full(1, F)],
        out_specs=_node_spec(F),
    )(agg2, y2, degp, b2p, w3p, b3p)

    return outp[:N, :N_CLASSES]
